# baseline (speedup 1.0000x reference)
import functools

import jax
import jax.numpy as jnp
from jax import lax
from jax.experimental import pallas as pl
from jax.experimental.pallas import tpu as pltpu
from jax.experimental.pallas import tpu_sc as plsc

N = 10000
NP = 10240
E = 320000
NW = 32
NSUB = 16
EPW = E // NW
C = 88
NC = -(-EPW // C)
EPP = NC * C
RPT = NP // NSUB
CW = 16

@functools.lru_cache(maxsize=None)
def _mesh():
    return plsc.VectorSubcoreMesh(
        core_axis_name="c", subcore_axis_name="s", num_cores=2,
        num_subcores=NSUB,
    )


@functools.lru_cache(maxsize=None)
def _make_sc_count():
    @functools.partial(
        pl.kernel,
        out_type=jax.ShapeDtypeStruct((NW, NP), jnp.float32),
        mesh=_mesh(),
        scratch_types=[
            pltpu.VMEM((NC, C), jnp.int32),
            pltpu.VMEM((NP,), jnp.float32),
        ],
        compiler_params=pltpu.CompilerParams(needs_layout_passes=False),
    )
    def count(dst_hbm, out_hbm, dst_v, deg_v):
        cid = lax.axis_index("c")
        sid = lax.axis_index("s")
        w = cid * NSUB + sid
        pltpu.sync_copy(dst_hbm.at[w], dst_v)

        def zero(i, carry):
            deg_v[pl.ds(i * 16, 16)] = jnp.zeros((16,), jnp.float32)
            return carry

        lax.fori_loop(0, NP // 16, zero, 0)
        ones16 = jnp.ones((16,), jnp.float32)

        def chunk(j, carry):
            def sub(k, c2):
                el = dst_v[j, pl.ds(k * 16, 16)]
                plsc.addupdate_scatter(deg_v, [el], ones16)
                return c2

            return lax.fori_loop(0, C // 16, sub, carry)

        lax.fori_loop(0, NC, chunk, 0)
        pltpu.sync_copy(deg_v, out_hbm.at[w])

    return count


@functools.lru_cache(maxsize=None)
def _make_sc_agg(D):
    @functools.partial(
        pl.kernel,
        out_type=jax.ShapeDtypeStruct((2, NP, D), jnp.float32),
        mesh=_mesh(),
        scratch_types=[
            pltpu.VMEM((NC, C), jnp.int32),
            pltpu.VMEM((NC, C), jnp.int32),
            pltpu.VMEM((C, D), jnp.float32),
            pltpu.VMEM_SHARED((NP, D), jnp.float32),
        ],
    )
    def agg(src_hbm, dst_hbm, tbl_hbm, out_hbm, src_v, dst_v, rows_v,
            acc_sh):
        cid = lax.axis_index("c")
        sid = lax.axis_index("s")
        w = cid * NSUB + sid
        pltpu.sync_copy(src_hbm.at[w], src_v)
        pltpu.sync_copy(dst_hbm.at[w], dst_v)
        pltpu.sync_copy(tbl_hbm.at[pl.ds(sid * RPT, RPT)],
                        acc_sh.at[pl.ds(sid * RPT, RPT)])
        plsc.subcore_barrier()

        def chunk(j, carry):
            pltpu.sync_copy(tbl_hbm.at[src_v.at[j]], rows_v)
            pltpu.sync_copy(rows_v, acc_sh.at[dst_v.at[j]], add=True)
            return carry

        lax.fori_loop(0, NC, chunk, 0)
        plsc.subcore_barrier()
        pltpu.sync_copy(acc_sh.at[pl.ds(sid * RPT, RPT)],
                        out_hbm.at[cid, pl.ds(sid * RPT, RPT)])

    return agg


BS = 2048


def _dinv_from_cnt(cnt_blk):
    deg = 1.0 + jnp.sum(cnt_blk, axis=1, keepdims=True)
    return lax.rsqrt(deg)


def _mm1_body(x_ref, cnt_ref, w_ref, out_ref):
    dinv = _dinv_from_cnt(cnt_ref[...])
    h = jnp.dot(x_ref[...], w_ref[...], preferred_element_type=jnp.float32)
    out_ref[...] = h * dinv


def _epi_mm2_body(s_ref, t_ref, cnt_ref, b_ref, w_ref, out_ref):
    dinv = _dinv_from_cnt(cnt_ref[...])
    s = s_ref[0] + s_ref[1] - t_ref[...]
    z = jnp.maximum(s * dinv + b_ref[...], 0.0)
    h2 = jnp.dot(z, w_ref[...], preferred_element_type=jnp.float32) * dinv
    out_ref[...] = jnp.concatenate(
        [h2, jnp.zeros_like(h2)], axis=1)


def _final_body(s_ref, t_ref, cnt_ref, b_ref, out_ref):
    dinv = _dinv_from_cnt(cnt_ref[...])
    s = s_ref[0][:, :64] + s_ref[1][:, :64] - t_ref[:, :64]
    out_ref[...] = s * dinv + b_ref[...]


def _row_spec(d):
    return pl.BlockSpec((BS, d), lambda i: (i, 0))


def _mm1(x, cnt, W1):
    grid = NP // BS
    return pl.pallas_call(
        _mm1_body,
        grid=(grid,),
        in_specs=[
            _row_spec(128),
            _row_spec(2 * CW),
            pl.BlockSpec((128, 128), lambda i: (0, 0)),
        ],
        out_specs=_row_spec(128),
        out_shape=jax.ShapeDtypeStruct((NP, 128), jnp.float32),
    )(x, cnt, W1)


def _epi_mm2(S, T1, cnt, b1, W2):
    grid = NP // BS
    return pl.pallas_call(
        _epi_mm2_body,
        grid=(grid,),
        in_specs=[
            pl.BlockSpec((2, BS, 128), lambda i: (0, i, 0)),
            _row_spec(128),
            _row_spec(2 * CW),
            pl.BlockSpec((1, 128), lambda i: (0, 0)),
            pl.BlockSpec((128, 64), lambda i: (0, 0)),
        ],
        out_specs=_row_spec(128),
        out_shape=jax.ShapeDtypeStruct((NP, 128), jnp.float32),
    )(S, T1, cnt, b1, W2)


def _final(S, T2, cnt, b2):
    grid = NP // BS
    return pl.pallas_call(
        _final_body,
        grid=(grid,),
        in_specs=[
            pl.BlockSpec((2, BS, 128), lambda i: (0, i, 0)),
            _row_spec(128),
            _row_spec(2 * CW),
            pl.BlockSpec((1, 64), lambda i: (0, 0)),
        ],
        out_specs=_row_spec(64),
        out_shape=jax.ShapeDtypeStruct((N, 64), jnp.float32),
    )(S, T2, cnt, b2)


def kernel(x, edge_index, W1, b1, W2, b2):
    src0 = edge_index[0].astype(jnp.int32).reshape(NW, EPW)
    dst0 = edge_index[1].astype(jnp.int32).reshape(NW, EPW)
    pad = EPP - EPW
    if pad:
        src = jnp.pad(src0, ((0, 0), (0, pad))).reshape(NW, NC, C)
        dump = (7 * jnp.arange(NW, dtype=jnp.int32)[:, None]
                + jnp.arange(pad, dtype=jnp.int32)) % (NP - N) + N
        dst = jnp.concatenate([dst0, dump], axis=1).reshape(NW, NC, C)
    else:
        src = src0.reshape(NW, NC, C)
        dst = dst0.reshape(NW, NC, C)

    cnt = _make_sc_count()(dst)
    cnt2 = cnt.T

    T1 = _mm1(x, cnt2, W1)
    S1 = _make_sc_agg(128)(src, dst, T1)
    T2 = _epi_mm2(S1, T1, cnt2, b1.reshape(1, 128), W2)
    S2 = _make_sc_agg(128)(src, dst, T2)
    return _final(S2, T2, cnt2, b2.reshape(1, 64))


# speedup vs baseline: 1.0884x; 1.0884x over previous
import functools

import jax
import jax.numpy as jnp
from jax import lax
from jax.experimental import pallas as pl
from jax.experimental.pallas import tpu as pltpu
from jax.experimental.pallas import tpu_sc as plsc

N = 10000
NP = 10240
E = 320000
NW = 32
NSUB = 16
EPW = E // NW
C = 80
NC = -(-EPW // C)
EPP = NC * C
RPT = NP // NSUB
CW = 16

@functools.lru_cache(maxsize=None)
def _mesh():
    return plsc.VectorSubcoreMesh(
        core_axis_name="c", subcore_axis_name="s", num_cores=2,
        num_subcores=NSUB,
    )


@functools.lru_cache(maxsize=None)
def _make_sc_count():
    @functools.partial(
        pl.kernel,
        out_type=jax.ShapeDtypeStruct((NW, NP), jnp.float32),
        mesh=_mesh(),
        scratch_types=[
            pltpu.VMEM((NC, C), jnp.int32),
            pltpu.VMEM((NP,), jnp.float32),
        ],
        compiler_params=pltpu.CompilerParams(needs_layout_passes=False),
    )
    def count(dst_hbm, out_hbm, dst_v, deg_v):
        cid = lax.axis_index("c")
        sid = lax.axis_index("s")
        w = cid * NSUB + sid
        pltpu.sync_copy(dst_hbm.at[w], dst_v)

        def zero(i, carry):
            deg_v[pl.ds(i * 16, 16)] = jnp.zeros((16,), jnp.float32)
            return carry

        lax.fori_loop(0, NP // 16, zero, 0)
        ones16 = jnp.ones((16,), jnp.float32)

        def chunk(j, carry):
            def sub(k, c2):
                el = dst_v[j, pl.ds(k * 16, 16)]
                plsc.addupdate_scatter(deg_v, [el], ones16)
                return c2

            return lax.fori_loop(0, C // 16, sub, carry)

        lax.fori_loop(0, NC, chunk, 0)
        pltpu.sync_copy(deg_v, out_hbm.at[w])

    return count


@functools.lru_cache(maxsize=None)
def _make_sc_agg(D):
    @functools.partial(
        pl.kernel,
        out_type=jax.ShapeDtypeStruct((2, NP, D), jnp.float32),
        mesh=_mesh(),
        scratch_types=[
            pltpu.VMEM((NC, C), jnp.int32),
            pltpu.VMEM((NC, C), jnp.int32),
            pltpu.VMEM((C, D), jnp.float32),
            pltpu.VMEM_SHARED((NP, D), jnp.float32),
        ],
    )
    def agg(src_hbm, dst_hbm, tbl_hbm, out_hbm, src_v, dst_v, rows_v,
            acc_sh):
        cid = lax.axis_index("c")
        sid = lax.axis_index("s")
        w = cid * NSUB + sid
        pltpu.sync_copy(src_hbm.at[w], src_v)
        pltpu.sync_copy(dst_hbm.at[w], dst_v)
        pltpu.sync_copy(tbl_hbm.at[pl.ds(sid * RPT, RPT)],
                        acc_sh.at[pl.ds(sid * RPT, RPT)])
        plsc.subcore_barrier()

        def chunk(j, carry):
            pltpu.sync_copy(tbl_hbm.at[src_v.at[j]], rows_v)
            pltpu.sync_copy(rows_v, acc_sh.at[dst_v.at[j]], add=True)
            return carry

        lax.fori_loop(0, NC, chunk, 0)
        plsc.subcore_barrier()
        pltpu.sync_copy(acc_sh.at[pl.ds(sid * RPT, RPT)],
                        out_hbm.at[cid, pl.ds(sid * RPT, RPT)])

    return agg


BS = 2048


def _dinv_from_cnt(cnt_blk):
    deg = 1.0 + jnp.sum(cnt_blk, axis=1, keepdims=True)
    return lax.rsqrt(deg)


def _mm1_body(x_ref, cnt_ref, w_ref, out_ref):
    dinv = _dinv_from_cnt(cnt_ref[...])
    h = jnp.dot(x_ref[...], w_ref[...], preferred_element_type=jnp.float32)
    out_ref[...] = h * dinv


def _epi_mm2_body(s_ref, t_ref, cnt_ref, b_ref, w_ref, out_ref):
    dinv = _dinv_from_cnt(cnt_ref[...])
    s = s_ref[0] + s_ref[1] - t_ref[...]
    z = jnp.maximum(s * dinv + b_ref[...], 0.0)
    h2 = jnp.dot(z, w_ref[...], preferred_element_type=jnp.float32) * dinv
    out_ref[...] = jnp.concatenate(
        [h2, jnp.zeros_like(h2)], axis=1)


def _final_body(s_ref, t_ref, cnt_ref, b_ref, out_ref):
    dinv = _dinv_from_cnt(cnt_ref[...])
    s = s_ref[0][:, :64] + s_ref[1][:, :64] - t_ref[:, :64]
    out_ref[...] = s * dinv + b_ref[...]


def _row_spec(d):
    return pl.BlockSpec((BS, d), lambda i: (i, 0))


def _mm1(x, cnt, W1):
    grid = NP // BS
    return pl.pallas_call(
        _mm1_body,
        grid=(grid,),
        in_specs=[
            _row_spec(128),
            _row_spec(2 * CW),
            pl.BlockSpec((128, 128), lambda i: (0, 0)),
        ],
        out_specs=_row_spec(128),
        out_shape=jax.ShapeDtypeStruct((NP, 128), jnp.float32),
    )(x, cnt, W1)


def _epi_mm2(S, T1, cnt, b1, W2):
    grid = NP // BS
    return pl.pallas_call(
        _epi_mm2_body,
        grid=(grid,),
        in_specs=[
            pl.BlockSpec((2, BS, 128), lambda i: (0, i, 0)),
            _row_spec(128),
            _row_spec(2 * CW),
            pl.BlockSpec((1, 128), lambda i: (0, 0)),
            pl.BlockSpec((128, 64), lambda i: (0, 0)),
        ],
        out_specs=_row_spec(128),
        out_shape=jax.ShapeDtypeStruct((NP, 128), jnp.float32),
    )(S, T1, cnt, b1, W2)


def _final(S, T2, cnt, b2):
    grid = NP // BS
    return pl.pallas_call(
        _final_body,
        grid=(grid,),
        in_specs=[
            pl.BlockSpec((2, BS, 128), lambda i: (0, i, 0)),
            _row_spec(128),
            _row_spec(2 * CW),
            pl.BlockSpec((1, 64), lambda i: (0, 0)),
        ],
        out_specs=_row_spec(64),
        out_shape=jax.ShapeDtypeStruct((N, 64), jnp.float32),
    )(S, T2, cnt, b2)


def kernel(x, edge_index, W1, b1, W2, b2):
    src0 = edge_index[0].astype(jnp.int32).reshape(NW, EPW)
    dst0 = edge_index[1].astype(jnp.int32).reshape(NW, EPW)
    pad = EPP - EPW
    if pad:
        src = jnp.pad(src0, ((0, 0), (0, pad))).reshape(NW, NC, C)
        dump = (7 * jnp.arange(NW, dtype=jnp.int32)[:, None]
                + jnp.arange(pad, dtype=jnp.int32)) % (NP - N) + N
        dst = jnp.concatenate([dst0, dump], axis=1).reshape(NW, NC, C)
    else:
        src = src0.reshape(NW, NC, C)
        dst = dst0.reshape(NW, NC, C)

    cnt = _make_sc_count()(dst)
    cnt2 = cnt.T

    T1 = _mm1(x, cnt2, W1)
    S1 = _make_sc_agg(128)(src, dst, T1)
    T2 = _epi_mm2(S1, T1, cnt2, b1.reshape(1, 128), W2)
    S2 = _make_sc_agg(128)(src, dst, T2)
    return _final(S2, T2, cnt2, b2.reshape(1, 64))


# speedup vs baseline: 1.0895x; 1.0010x over previous
import functools

import jax
import jax.numpy as jnp
from jax import lax
from jax.experimental import pallas as pl
from jax.experimental.pallas import tpu as pltpu
from jax.experimental.pallas import tpu_sc as plsc

N = 10000
NP = 10240
E = 320000
NW = 32
NSUB = 16
EPW = E // NW
C = 80
NC = -(-EPW // C)
EPP = NC * C
RPT = NP // NSUB

@functools.lru_cache(maxsize=None)
def _mesh():
    return plsc.VectorSubcoreMesh(
        core_axis_name="c", subcore_axis_name="s", num_cores=2,
        num_subcores=NSUB,
    )


@functools.lru_cache(maxsize=None)
def _make_sc_count():
    @functools.partial(
        pl.kernel,
        out_type=jax.ShapeDtypeStruct((NW, NP), jnp.float32),
        mesh=_mesh(),
        scratch_types=[
            pltpu.VMEM((NC, C), jnp.int32),
            pltpu.VMEM((NP,), jnp.float32),
        ],
        compiler_params=pltpu.CompilerParams(needs_layout_passes=False),
    )
    def count(dst_hbm, out_hbm, dst_v, deg_v):
        cid = lax.axis_index("c")
        sid = lax.axis_index("s")
        w = cid * NSUB + sid
        pltpu.sync_copy(dst_hbm.at[w], dst_v)

        def zero(i, carry):
            deg_v[pl.ds(i * 16, 16)] = jnp.zeros((16,), jnp.float32)
            return carry

        lax.fori_loop(0, NP // 16, zero, 0)
        ones16 = jnp.ones((16,), jnp.float32)

        def chunk(j, carry):
            def sub(k, c2):
                el = dst_v[j, pl.ds(k * 16, 16)]
                plsc.addupdate_scatter(deg_v, [el], ones16)
                return c2

            return lax.fori_loop(0, C // 16, sub, carry)

        lax.fori_loop(0, NC, chunk, 0)
        pltpu.sync_copy(deg_v, out_hbm.at[w])

    return count


@functools.lru_cache(maxsize=None)
def _make_sc_agg(D):
    @functools.partial(
        pl.kernel,
        out_type=jax.ShapeDtypeStruct((2, NP, D), jnp.float32),
        mesh=_mesh(),
        scratch_types=[
            pltpu.VMEM((NC, C), jnp.int32),
            pltpu.VMEM((NC, C), jnp.int32),
            pltpu.VMEM((C, D), jnp.float32),
            pltpu.VMEM_SHARED((NP, D), jnp.float32),
        ],
    )
    def agg(src_hbm, dst_hbm, tbl_hbm, out_hbm, src_v, dst_v, rows_v,
            acc_sh):
        cid = lax.axis_index("c")
        sid = lax.axis_index("s")
        w = cid * NSUB + sid
        pltpu.sync_copy(src_hbm.at[w], src_v)
        pltpu.sync_copy(dst_hbm.at[w], dst_v)
        pltpu.sync_copy(tbl_hbm.at[pl.ds(sid * RPT, RPT)],
                        acc_sh.at[pl.ds(sid * RPT, RPT)])
        plsc.subcore_barrier()

        def chunk(j, carry):
            pltpu.sync_copy(tbl_hbm.at[src_v.at[j]], rows_v)
            pltpu.sync_copy(rows_v, acc_sh.at[dst_v.at[j]], add=True)
            return carry

        lax.fori_loop(0, NC, chunk, 0)
        plsc.subcore_barrier()
        pltpu.sync_copy(acc_sh.at[pl.ds(sid * RPT, RPT)],
                        out_hbm.at[cid, pl.ds(sid * RPT, RPT)])

    return agg


BS = 2048


def _dinv_from_cnt(cnt_blk):
    deg = 1.0 + jnp.sum(cnt_blk, axis=1, keepdims=True)
    return lax.rsqrt(deg)


def _mm1_body(x_ref, cnt_ref, w_ref, out_ref):
    dinv = _dinv_from_cnt(cnt_ref[...])
    h = jnp.dot(x_ref[...], w_ref[...], preferred_element_type=jnp.float32)
    out_ref[...] = h * dinv


def _epi_mm2_body(s_ref, t_ref, cnt_ref, b_ref, w_ref, out_ref):
    dinv = _dinv_from_cnt(cnt_ref[...])
    s = s_ref[0] + s_ref[1] - t_ref[...]
    z = jnp.maximum(s * dinv + b_ref[...], 0.0)
    h2 = jnp.dot(z, w_ref[...], preferred_element_type=jnp.float32) * dinv
    out_ref[...] = jnp.concatenate(
        [h2, jnp.zeros_like(h2)], axis=1)


def _final_body(s_ref, t_ref, cnt_ref, b_ref, out_ref):
    dinv = _dinv_from_cnt(cnt_ref[...])
    s = s_ref[0][:, :64] + s_ref[1][:, :64] - t_ref[:, :64]
    out_ref[...] = s * dinv + b_ref[...]


def _row_spec(d):
    return pl.BlockSpec((BS, d), lambda i: (i, 0))


def _mm1(x, cnt, W1):
    grid = NP // BS
    return pl.pallas_call(
        _mm1_body,
        grid=(grid,),
        in_specs=[
            _row_spec(128),
            _row_spec(NW),
            pl.BlockSpec((128, 128), lambda i: (0, 0)),
        ],
        out_specs=_row_spec(128),
        out_shape=jax.ShapeDtypeStruct((NP, 128), jnp.float32),
    )(x, cnt, W1)


def _epi_mm2(S, T1, cnt, b1, W2):
    grid = NP // BS
    return pl.pallas_call(
        _epi_mm2_body,
        grid=(grid,),
        in_specs=[
            pl.BlockSpec((2, BS, 128), lambda i: (0, i, 0)),
            _row_spec(128),
            _row_spec(NW),
            pl.BlockSpec((1, 128), lambda i: (0, 0)),
            pl.BlockSpec((128, 64), lambda i: (0, 0)),
        ],
        out_specs=_row_spec(128),
        out_shape=jax.ShapeDtypeStruct((NP, 128), jnp.float32),
    )(S, T1, cnt, b1, W2)


def _final(S, T2, cnt, b2):
    grid = NP // BS
    return pl.pallas_call(
        _final_body,
        grid=(grid,),
        in_specs=[
            pl.BlockSpec((2, BS, 128), lambda i: (0, i, 0)),
            _row_spec(128),
            _row_spec(NW),
            pl.BlockSpec((1, 64), lambda i: (0, 0)),
        ],
        out_specs=_row_spec(64),
        out_shape=jax.ShapeDtypeStruct((N, 64), jnp.float32),
    )(S, T2, cnt, b2)


def kernel(x, edge_index, W1, b1, W2, b2):
    src0 = edge_index[0].astype(jnp.int32).reshape(NW, EPW)
    dst0 = edge_index[1].astype(jnp.int32).reshape(NW, EPW)
    pad = EPP - EPW
    if pad:
        src = jnp.pad(src0, ((0, 0), (0, pad))).reshape(NW, NC, C)
        dump = (7 * jnp.arange(NW, dtype=jnp.int32)[:, None]
                + jnp.arange(pad, dtype=jnp.int32)) % (NP - N) + N
        dst = jnp.concatenate([dst0, dump], axis=1).reshape(NW, NC, C)
    else:
        src = src0.reshape(NW, NC, C)
        dst = dst0.reshape(NW, NC, C)

    cnt = _make_sc_count()(dst)
    cnt2 = cnt.T

    T1 = _mm1(x, cnt2, W1)
    S1 = _make_sc_agg(128)(src, dst, T1)
    T2 = _epi_mm2(S1, T1, cnt2, b1.reshape(1, 128), W2)
    S2 = _make_sc_agg(128)(src, dst, T2)
    return _final(S2, T2, cnt2, b2.reshape(1, 64))
